# baseline (speedup 1.0000x reference)
import functools

import jax
import jax.numpy as jnp
from jax import lax
from jax.experimental import pallas as pl
from jax.experimental.pallas import tpu as pltpu
from jax.experimental.pallas import tpu_sc as plsc

DIM = 64
NNEG = 20
LANES = 16
NSLICE = DIM // LANES
NPAD = 32


def _make_sc_scores(vocab, batch, dim, nneg):
    info = plsc.get_sparse_core_info()
    nw = info.num_cores * info.num_subcores
    bw = batch // nw
    egrp = 16
    ngrp = bw // egrp
    rows_g = egrp * nneg
    mesh = plsc.VectorSubcoreMesh(core_axis_name="c", subcore_axis_name="s")

    @functools.partial(
        pl.kernel,
        out_type=[
            jax.ShapeDtypeStruct((batch,), jnp.float32),
            jax.ShapeDtypeStruct((batch, NPAD), jnp.float32),
        ],
        mesh=mesh,
        scratch_types=[
            pltpu.VMEM((bw,), jnp.int32),
            pltpu.VMEM((bw,), jnp.int32),
            pltpu.VMEM((bw * nneg,), jnp.int32),
            pltpu.VMEM((bw, dim), jnp.float32),
            pltpu.VMEM((bw, dim), jnp.float32),
            pltpu.VMEM((rows_g, dim), jnp.float32),
            pltpu.VMEM((bw,), jnp.float32),
            pltpu.VMEM((bw, NPAD), jnp.float32),
            pltpu.SemaphoreType.DMA,
        ],
        compiler_params=pltpu.CompilerParams(use_tc_tiling_on_sc=False),
    )
    def sc_scores(c_hbm, p_hbm, n_hbm, in_hbm, out_hbm, pos_o, neg_o,
                  cidx, pidx, nidx, crow, prow, nrow, ps, ns, sem):
        wid = lax.axis_index("s") * info.num_cores + lax.axis_index("c")
        base = wid * bw
        lane = lax.iota(jnp.int32, LANES)

        pltpu.sync_copy(c_hbm.at[pl.ds(base, bw)], cidx)
        pltpu.sync_copy(p_hbm.at[pl.ds(base, bw)], pidx)
        pltpu.sync_copy(n_hbm.at[pl.ds(base * nneg, bw * nneg)], nidx)

        pltpu.async_copy(in_hbm.at[cidx], crow, sem).wait()
        pltpu.async_copy(out_hbm.at[pidx], prow, sem).wait()

        def dot_row(a_ref, i, b_ref, j):
            acc = a_ref[i, pl.ds(0, LANES)] * b_ref[j, pl.ds(0, LANES)]
            for k in range(1, NSLICE):
                acc += (a_ref[i, pl.ds(k * LANES, LANES)]
                        * b_ref[j, pl.ds(k * LANES, LANES)])
            for sh in (8, 4, 2, 1):
                acc = acc + jnp.take(acc, lane ^ sh)
            return acc

        def pos_body(v, _):
            vec = jnp.zeros((LANES,), jnp.float32)
            for e in range(LANES):
                s = dot_row(crow, v * LANES + e, prow, v * LANES + e)
                vec = jnp.where(lane == e, s, vec)
            ps[pl.ds(v * LANES, LANES)] = vec
            return 0
        lax.fori_loop(0, bw // LANES, pos_body, 0)

        def grp_body(g, _):
            pltpu.async_copy(
                out_hbm.at[nidx.at[pl.ds(g * rows_g, rows_g)]], nrow, sem
            ).wait()

            def elem_body(e, _):
                i = g * egrp + e
                lo = jnp.zeros((LANES,), jnp.float32)
                hi = jnp.zeros((LANES,), jnp.float32)
                for j in range(nneg):
                    s = dot_row(crow, i, nrow, e * nneg + j)
                    if j < LANES:
                        lo = jnp.where(lane == j, s, lo)
                    else:
                        hi = jnp.where(lane == j - LANES, s, hi)
                ns[i, pl.ds(0, LANES)] = lo
                ns[i, pl.ds(LANES, LANES)] = hi
                return 0
            lax.fori_loop(0, egrp, elem_body, 0)
            return 0
        lax.fori_loop(0, ngrp, grp_body, 0)

        pltpu.sync_copy(ps, pos_o.at[pl.ds(base, bw)])
        pltpu.sync_copy(ns, neg_o.at[pl.ds(base, bw)])

    return sc_scores


def _loss_body(pos_ref, neg_ref, out_ref, *, batch, nneg):
    def ls(x):
        return jnp.minimum(x, 0.0) - jnp.log1p(jnp.exp(-jnp.abs(x)))
    col = lax.broadcasted_iota(jnp.int32, neg_ref.shape, 1)
    neg_ls = jnp.where(col < nneg, ls(-neg_ref[...]), 0.0)
    total = jnp.sum(ls(pos_ref[...])) + jnp.sum(neg_ls)
    out_ref[...] = jnp.broadcast_to(-total / batch, (1, 1))


def kernel(center_words, context_words, negative_words, in_emb, out_emb):
    vocab, dim = in_emb.shape
    batch = center_words.shape[0]
    nneg = negative_words.shape[1]

    sc_scores = _make_sc_scores(vocab, batch, dim, nneg)
    pos_s, neg_s = sc_scores(
        center_words, context_words, negative_words.reshape(-1),
        in_emb, out_emb)

    pos2 = pos_s.reshape(batch // 128, 128)
    loss = pl.pallas_call(
        functools.partial(_loss_body, batch=batch, nneg=nneg),
        out_shape=jax.ShapeDtypeStruct((1, 1), jnp.float32),
    )(pos2, neg_s)
    return loss.reshape(())
